# baseline (speedup 1.0000x reference)
import functools

import jax
import jax.numpy as jnp
from jax import lax
from jax.experimental import pallas as pl
from jax.experimental.pallas import tpu as pltpu
from jax.experimental.pallas import tpu_sc as plsc

BATCH = 16384
EMB = 32
GATHER_WINDOW = 128


def _sc_gather(user_idx, movie_idx, user_table, movie_table):
    b = user_idx.shape[0]
    uidx = user_idx.reshape(1, b).astype(jnp.int32)
    midx = movie_idx.reshape(1, b).astype(jnp.int32)
    mesh = plsc.VectorSubcoreMesh(core_axis_name="core", subcore_axis_name="subcore")

    @functools.partial(
        pl.kernel,
        out_type=(
            jax.ShapeDtypeStruct((b, EMB), jnp.float32),
            jax.ShapeDtypeStruct((b, EMB), jnp.float32),
        ),
        mesh=mesh,
        compiler_params=pltpu.CompilerParams(use_tc_tiling_on_sc=False),
    )
    def gather_kernel(utab_hbm, mtab_hbm, uidx_hbm, midx_hbm, uout_hbm, mout_hbm):
        def body(ui_v, mi_v, uo_v, mo_v):
            pltpu.sync_copy(utab_hbm.at[ui_v.at[0]], uo_v)
            pltpu.sync_copy(mtab_hbm.at[mi_v.at[0]], mo_v)

        pltpu.emit_pipeline(
            body,
            grid=(b // GATHER_WINDOW,),
            in_specs=[
                pl.BlockSpec((1, GATHER_WINDOW), lambda i: (0, i)),
                pl.BlockSpec((1, GATHER_WINDOW), lambda i: (0, i)),
            ],
            out_specs=[
                pl.BlockSpec((GATHER_WINDOW, EMB), lambda i: (i, 0)),
                pl.BlockSpec((GATHER_WINDOW, EMB), lambda i: (i, 0)),
            ],
            core_axis_name=("core", "subcore"),
            dimension_semantics=(pltpu.PARALLEL,),
        )(uidx_hbm, midx_hbm, uout_hbm, mout_hbm)

    return gather_kernel(user_table, movie_table, uidx, midx)


def _mlp_body(u_ref, m_ref, w1u_ref, w1m_ref, b1_ref, w2_ref, b2_ref,
              w3_ref, b3_ref, o_ref):
    dn = (((1,), (1,)), ((), ()))
    hp = jax.lax.Precision.HIGHEST
    u = u_ref[...]
    m = m_ref[...]
    h = lax.dot_general(u, w1u_ref[...], dn, precision=hp,
                        preferred_element_type=jnp.float32)
    h += lax.dot_general(m, w1m_ref[...], dn, precision=hp,
                         preferred_element_type=jnp.float32)
    h = jnp.maximum(h + b1_ref[...][None, :], 0.0)
    h = lax.dot_general(h, w2_ref[...], dn, precision=hp,
                        preferred_element_type=jnp.float32)
    h = jnp.maximum(h + b2_ref[...][None, :], 0.0)
    o_ref[...] = jnp.sum(h * w3_ref[...][0][None, :], axis=1) + b3_ref[...]


def _tc_mlp(u_vec, m_vec, W1, b1, W2, b2, W3, b3):
    b = u_vec.shape[0]
    bm = 2048
    w1u = W1[:, :EMB]
    w1m = W1[:, EMB:]
    grid = (b // bm,)
    return pl.pallas_call(
        _mlp_body,
        grid=grid,
        in_specs=[
            pl.BlockSpec((bm, EMB), lambda i: (i, 0)),
            pl.BlockSpec((bm, EMB), lambda i: (i, 0)),
            pl.BlockSpec(w1u.shape, lambda i: (0, 0)),
            pl.BlockSpec(w1m.shape, lambda i: (0, 0)),
            pl.BlockSpec(b1.shape, lambda i: (0,)),
            pl.BlockSpec(W2.shape, lambda i: (0, 0)),
            pl.BlockSpec(b2.shape, lambda i: (0,)),
            pl.BlockSpec(W3.shape, lambda i: (0, 0)),
            pl.BlockSpec(b3.shape, lambda i: (0,)),
        ],
        out_specs=pl.BlockSpec((bm,), lambda i: (i,)),
        out_shape=jax.ShapeDtypeStruct((b,), jnp.float32),
    )(u_vec, m_vec, w1u, w1m, b1, W2, b2, W3, b3)


def kernel(user_idx, movie_idx, user_table, movie_table, W1, b1, W2, b2, W3, b3):
    u_vec, m_vec = _sc_gather(user_idx, movie_idx, user_table, movie_table)
    return _tc_mlp(u_vec, m_vec, W1, b1, W2, b2, W3, b3)
